# baseline (speedup 1.0000x reference)
import functools

import jax
import jax.numpy as jnp
from jax import lax
from jax.experimental import pallas as pl
from jax.experimental.pallas import tpu as pltpu
from jax.experimental.pallas import tpu_sc as plsc

_N = 4096
_NC, _NS, _L = 2, 16, 16
_NW = _NC * _NS
_RPW = _N // _NW
_K = 8
_IDX_PAD = 256

_BETA = 0.9


def _cumsum16(v):
    io = lax.iota(jnp.int32, _L)
    for s in (1, 2, 4, 8):
        shifted = v.at[jnp.maximum(io - s, 0)].get(mode="promise_in_bounds")
        v = v + jnp.where(io >= s, shifted, 0)
    return v


def _sc_body(spk_hbm, w_hbm, part_hbm, mask_v, idx_v, rows_v, acc_v, cnt_v, sem):
    wid = lax.axis_index("s") * _NC + lax.axis_index("c")
    base = wid * _RPW

    pltpu.sync_copy(spk_hbm.at[pl.ds(base, _RPW)], mask_v)

    for i in range(_IDX_PAD // _L):
        idx_v[pl.ds(i * _L, _L)] = jnp.zeros((_L,), jnp.int32)

    def _zero(i, carry):
        off = pl.multiple_of(i * _L, _L)
        acc_v[pl.ds(off, _L)] = jnp.zeros((_L,), jnp.float32)
        return carry

    lax.fori_loop(0, _N // _L, _zero, 0)

    last = jnp.full((_L,), _L - 1, jnp.int32)
    off_vec = jnp.zeros((_L,), jnp.int32)
    for i in range(_RPW // _L):
        mv = mask_v[pl.ds(i * _L, _L)]
        m = mv > 0
        cs = _cumsum16(mv)
        pos = off_vec + cs - 1
        idxvec = base + i * _L + lax.iota(jnp.int32, _L)
        plsc.store_scatter(idx_v, [pos], idxvec, mask=m)
        off_vec = off_vec + cs.at[last].get(mode="promise_in_bounds")
    count = off_vec[0]

    n_chunks = (count + _K - 1) // _K

    def _chunk(t, carry):
        tbase = pl.multiple_of(t * _K, _K)
        pltpu.async_copy(w_hbm.at[idx_v.at[pl.ds(tbase, _K)]], rows_v, sem).wait()
        for j in range(_K):
            vf = (tbase + j < count).astype(jnp.float32)

            def _acc(ci, c2):
                o = pl.multiple_of(ci * _L, _L)
                acc_v[pl.ds(o, _L)] = (
                    acc_v[pl.ds(o, _L)] + rows_v[j, pl.ds(o, _L)] * vf
                )
                return c2

            lax.fori_loop(0, _N // _L, _acc, 0)
        return carry

    lax.fori_loop(0, n_chunks, _chunk, 0)

    pltpu.sync_copy(acc_v, part_hbm.at[wid])


def _epilogue_body(part_ref, x_ref, act_ref, thr_ref, out_ref):
    lat = jnp.sum(part_ref[...], axis=0)
    v = _BETA * act_ref[...] + x_ref[...] + lat
    out_ref[...] = (v > thr_ref[...]).astype(jnp.float32)


@jax.jit
def kernel(x, activation, spikes, threshold, freq, lateral_weights):
    del freq
    spk_i32 = spikes.reshape(-1).astype(jnp.int32)

    mesh = plsc.VectorSubcoreMesh(
        core_axis_name="c", subcore_axis_name="s", num_cores=_NC, num_subcores=_NS
    )
    sc_kernel = pl.kernel(
        _sc_body,
        out_type=jax.ShapeDtypeStruct((_NW, _N), jnp.float32),
        mesh=mesh,
        scratch_types=[
            pltpu.VMEM((_RPW,), jnp.int32),
            pltpu.VMEM((_IDX_PAD,), jnp.int32),
            pltpu.VMEM((_K, _N), jnp.float32),
            pltpu.VMEM((_N,), jnp.float32),
            pltpu.VMEM((_L,), jnp.int32),
            pltpu.SemaphoreType.DMA,
        ],
        compiler_params=pltpu.CompilerParams(needs_layout_passes=False),
    )
    partials = sc_kernel(spk_i32, lateral_weights)

    outf = pl.pallas_call(
        _epilogue_body,
        out_shape=jax.ShapeDtypeStruct((_N,), jnp.float32),
    )(partials, x.reshape(-1), activation.reshape(-1), threshold.reshape(-1))
    return outf.astype(jnp.bool_).reshape(x.shape)


# speedup vs baseline: 1.8174x; 1.8174x over previous
import functools

import jax
import jax.numpy as jnp
from jax import lax
from jax.experimental import pallas as pl
from jax.experimental.pallas import tpu as pltpu
from jax.experimental.pallas import tpu_sc as plsc

_N = 4096
_NC, _NS, _L = 2, 16, 16
_NW = _NC * _NS
_RPW = _N // _NW
_K = 8
_IDX_PAD = 256

_BETA = 0.9


def _cumsum16(v):
    io = lax.iota(jnp.int32, _L)
    for s in (1, 2, 4, 8):
        shifted = v.at[jnp.maximum(io - s, 0)].get(mode="promise_in_bounds")
        v = v + jnp.where(io >= s, shifted, 0)
    return v


def _sc_body(spk_hbm, w_hbm, part_hbm, mask_v, idx_v, rows_v, acc_v, cnt_v, sem):
    wid = lax.axis_index("s") * _NC + lax.axis_index("c")
    base = wid * _RPW

    pltpu.sync_copy(spk_hbm.at[pl.ds(base, _RPW)], mask_v)

    for i in range(_IDX_PAD // _L):
        idx_v[pl.ds(i * _L, _L)] = jnp.zeros((_L,), jnp.int32)

    def _zero(i, carry):
        off = pl.multiple_of(i * _L, _L)
        acc_v[pl.ds(off, _L)] = jnp.zeros((_L,), jnp.float32)
        return carry

    lax.fori_loop(0, _N // _L, _zero, 0)

    last = jnp.full((_L,), _L - 1, jnp.int32)
    off_vec = jnp.zeros((_L,), jnp.int32)
    for i in range(_RPW // _L):
        mv = mask_v[pl.ds(i * _L, _L)]
        m = mv > 0
        cs = _cumsum16(mv)
        pos = off_vec + cs - 1
        idxvec = base + i * _L + lax.iota(jnp.int32, _L)
        plsc.store_scatter(idx_v, [pos], idxvec, mask=m)
        off_vec = off_vec + cs.at[last].get(mode="promise_in_bounds")
    count = off_vec[0]

    n_chunks = (count + _K - 1) // _K

    def _chunk(t, carry):
        tbase = pl.multiple_of(t * _K, _K)
        pltpu.async_copy(w_hbm.at[idx_v.at[pl.ds(tbase, _K)]], rows_v, sem).wait()
        vf = [(tbase + j < count).astype(jnp.float32) for j in range(_K)]

        def _acc(ci, c2):
            o = pl.multiple_of(ci * _L, _L)
            a = acc_v[pl.ds(o, _L)]
            for j in range(_K):
                a = a + rows_v[j, pl.ds(o, _L)] * vf[j]
            acc_v[pl.ds(o, _L)] = a
            return c2

        lax.fori_loop(0, _N // _L, _acc, 0)
        return carry

    lax.fori_loop(0, n_chunks, _chunk, 0)

    pltpu.sync_copy(acc_v, part_hbm.at[wid])


def _epilogue_body(part_ref, x_ref, act_ref, thr_ref, out_ref):
    lat = jnp.sum(part_ref[...], axis=0)
    v = _BETA * act_ref[...] + x_ref[...] + lat
    out_ref[...] = (v > thr_ref[...]).astype(jnp.float32)


@jax.jit
def kernel(x, activation, spikes, threshold, freq, lateral_weights):
    del freq
    spk_i32 = spikes.reshape(-1).astype(jnp.int32)

    mesh = plsc.VectorSubcoreMesh(
        core_axis_name="c", subcore_axis_name="s", num_cores=_NC, num_subcores=_NS
    )
    sc_kernel = pl.kernel(
        _sc_body,
        out_type=jax.ShapeDtypeStruct((_NW, _N), jnp.float32),
        mesh=mesh,
        scratch_types=[
            pltpu.VMEM((_RPW,), jnp.int32),
            pltpu.VMEM((_IDX_PAD,), jnp.int32),
            pltpu.VMEM((_K, _N), jnp.float32),
            pltpu.VMEM((_N,), jnp.float32),
            pltpu.VMEM((_L,), jnp.int32),
            pltpu.SemaphoreType.DMA,
        ],
        compiler_params=pltpu.CompilerParams(needs_layout_passes=False),
    )
    partials = sc_kernel(spk_i32, lateral_weights)

    outf = pl.pallas_call(
        _epilogue_body,
        out_shape=jax.ShapeDtypeStruct((_N,), jnp.float32),
    )(partials, x.reshape(-1), activation.reshape(-1), threshold.reshape(-1))
    return outf.astype(jnp.bool_).reshape(x.shape)


# speedup vs baseline: 2.1629x; 1.1901x over previous
import functools

import jax
import jax.numpy as jnp
from jax import lax
from jax.experimental import pallas as pl
from jax.experimental.pallas import tpu as pltpu
from jax.experimental.pallas import tpu_sc as plsc

_N = 4096
_NC, _NS, _L = 2, 16, 16
_NW = _NC * _NS
_RPW = _N // _NW
_K = 8
_IDX_PAD = 256

_BETA = 0.9


def _cumsum16(v):
    io = lax.iota(jnp.int32, _L)
    for s in (1, 2, 4, 8):
        shifted = v.at[jnp.maximum(io - s, 0)].get(mode="promise_in_bounds")
        v = v + jnp.where(io >= s, shifted, 0)
    return v


def _sc_body(spk_hbm, w_hbm, part_hbm, mask_v, idx_v, rows_a, rows_b, acc_v, sem_a, sem_b):
    wid = lax.axis_index("s") * _NC + lax.axis_index("c")
    base = wid * _RPW

    pltpu.sync_copy(spk_hbm.at[pl.ds(base, _RPW)], mask_v)

    for i in range(_IDX_PAD // _L):
        idx_v[pl.ds(i * _L, _L)] = jnp.zeros((_L,), jnp.int32)

    def _zero(i, carry):
        off = pl.multiple_of(i * _L, _L)
        acc_v[pl.ds(off, _L)] = jnp.zeros((_L,), jnp.float32)
        return carry

    lax.fori_loop(0, _N // _L, _zero, 0)

    last = jnp.full((_L,), _L - 1, jnp.int32)
    off_vec = jnp.zeros((_L,), jnp.int32)
    for i in range(_RPW // _L):
        mv = mask_v[pl.ds(i * _L, _L)]
        m = mv > 0
        cs = _cumsum16(mv)
        pos = off_vec + cs - 1
        idxvec = base + i * _L + lax.iota(jnp.int32, _L)
        plsc.store_scatter(idx_v, [pos], idxvec, mask=m)
        off_vec = off_vec + cs.at[last].get(mode="promise_in_bounds")
    count = off_vec[0]

    n_chunks = (count + _K - 1) // _K
    n_outer = (n_chunks + 1) // 2

    def _gather(t, buf, s):
        tb = pl.multiple_of(t * _K, _K)
        return pltpu.make_async_copy(w_hbm.at[idx_v.at[pl.ds(tb, _K)]], buf, s)

    def _compute(t, buf):
        tbase = t * _K
        vf = [(tbase + j < count).astype(jnp.float32) for j in range(_K)]

        def _acc(ci, c2):
            o = pl.multiple_of(ci * _L, _L)
            a = acc_v[pl.ds(o, _L)]
            for j in range(_K):
                a = a + buf[j, pl.ds(o, _L)] * vf[j]
            acc_v[pl.ds(o, _L)] = a
            return c2

        lax.fori_loop(0, _N // _L, _acc, 0)

    @pl.when(n_chunks > 0)
    def _prime():
        _gather(0, rows_a, sem_a).start()

    def _outer(u, carry):
        t0 = u * 2

        @pl.when(t0 + 1 < n_chunks)
        def _start_b():
            _gather(t0 + 1, rows_b, sem_b).start()

        _gather(t0, rows_a, sem_a).wait()
        _compute(t0, rows_a)

        @pl.when(t0 + 2 < n_chunks)
        def _start_a():
            _gather(t0 + 2, rows_a, sem_a).start()

        @pl.when(t0 + 1 < n_chunks)
        def _do_b():
            _gather(t0 + 1, rows_b, sem_b).wait()
            _compute(t0 + 1, rows_b)

        return carry

    lax.fori_loop(0, n_outer, _outer, 0)

    pltpu.sync_copy(acc_v, part_hbm.at[wid])


def _epilogue_body(part_ref, x_ref, act_ref, thr_ref, out_ref):
    lat = jnp.sum(part_ref[...], axis=0)
    v = _BETA * act_ref[...] + x_ref[...] + lat
    out_ref[...] = (v > thr_ref[...]).astype(jnp.float32)


@jax.jit
def kernel(x, activation, spikes, threshold, freq, lateral_weights):
    del freq
    spk_i32 = spikes.reshape(-1).astype(jnp.int32)

    mesh = plsc.VectorSubcoreMesh(
        core_axis_name="c", subcore_axis_name="s", num_cores=_NC, num_subcores=_NS
    )
    sc_kernel = pl.kernel(
        _sc_body,
        out_type=jax.ShapeDtypeStruct((_NW, _N), jnp.float32),
        mesh=mesh,
        scratch_types=[
            pltpu.VMEM((_RPW,), jnp.int32),
            pltpu.VMEM((_IDX_PAD,), jnp.int32),
            pltpu.VMEM((_K, _N), jnp.float32),
            pltpu.VMEM((_K, _N), jnp.float32),
            pltpu.VMEM((_N,), jnp.float32),
            pltpu.SemaphoreType.DMA,
            pltpu.SemaphoreType.DMA,
        ],
        compiler_params=pltpu.CompilerParams(needs_layout_passes=False),
    )
    partials = sc_kernel(spk_i32, lateral_weights)

    outf = pl.pallas_call(
        _epilogue_body,
        out_shape=jax.ShapeDtypeStruct((_N,), jnp.float32),
    )(partials, x.reshape(-1), activation.reshape(-1), threshold.reshape(-1))
    return outf.astype(jnp.bool_).reshape(x.shape)
